# baseline (device time: 45963 ns/iter reference)
import jax
import jax.numpy as jnp
from jax import lax
from jax.experimental import pallas as pl
from jax.experimental.pallas import tpu as pltpu

N_DEV = 16
C = 32
MESH = pl.DeviceIdType.MESH


def kernel(x, assign, W1, W2):
    T, D = x.shape
    E, _, F = W1.shape
    R = N_DEV * C

    def body(x_ref, a_ref, w1_ref, w2_ref, out_ref,
             w1b, w2b, selbig, xcbig, xbuf, ybuf, ybig,
             xsend, xrecv, ysend, yrecv):
        my = lax.axis_index("i")

        barrier = pltpu.get_barrier_semaphore()
        for d in range(1, N_DEV):
            peer = lax.rem(my + d, N_DEV)
            pl.semaphore_signal(barrier, inc=1, device_id=(peer,),
                                device_id_type=MESH)
        pl.semaphore_wait(barrier, N_DEV - 1)

        w1b[...] = w1_ref[...].astype(jnp.bfloat16)
        w2b[...] = w2_ref[...].astype(jnp.bfloat16)

        a_row = a_ref[...]
        gids = lax.broadcasted_iota(jnp.int32, (2 * N_DEV, T), 0)
        onehot = (a_row == gids).astype(jnp.float32)
        ti = lax.broadcasted_iota(jnp.int32, (T, T), 0)
        tj = lax.broadcasted_iota(jnp.int32, (T, T), 1)
        ltri = (ti <= tj).astype(jnp.float32)
        cum = jnp.dot(onehot, ltri, preferred_element_type=jnp.float32)
        rank_row = jnp.sum(onehot * (cum - 1.0), axis=0, keepdims=True)

        rows_d = lax.broadcasted_iota(
            jnp.int32, (N_DEV, C, T), 0).reshape(R, T)
        rows_k = lax.broadcasted_iota(
            jnp.int32, (N_DEV, C, T), 1).reshape(R, T)
        for e in range(E):
            dst_expert = 2 * lax.rem(my + rows_d, N_DEV) + e
            sel = (a_row == dst_expert) & (rank_row == rows_k.astype(jnp.float32))
            selbig[e] = sel.astype(jnp.bfloat16)

        xb = x_ref[...].astype(jnp.bfloat16)
        for e in range(E):
            xcbig[e] = jnp.dot(selbig[e], xb,
                               preferred_element_type=jnp.float32
                               ).astype(jnp.bfloat16)

        xbuf[:, 0] = xcbig[:, 0:C, :]
        xd = []
        for d in range(1, N_DEV):
            r = pltpu.make_async_remote_copy(
                src_ref=xcbig.at[:, d * C:(d + 1) * C, :],
                dst_ref=xbuf.at[:, d],
                send_sem=xsend.at[d], recv_sem=xrecv.at[d],
                device_id=(lax.rem(my + d, N_DEV),), device_id_type=MESH)
            r.start()
            xd.append(r)

        yd = []
        for lo, hi in ((0, N_DEV // 2), (N_DEV // 2, N_DEV)):
            for r in xd[max(lo, 1) - 1:hi - 1]:
                r.wait_recv()
            for e in range(E):
                xh = xbuf[e, lo:hi].reshape((hi - lo) * C, D)
                h1 = jnp.maximum(
                    jnp.dot(xh, w1b[e], preferred_element_type=jnp.float32),
                    0.0).astype(jnp.bfloat16)
                ybig[e, lo * C:hi * C] = jnp.dot(
                    h1, w2b[e], preferred_element_type=jnp.float32
                ).astype(jnp.bfloat16)
            if lo == 0:
                ybuf[:, 0] = ybig[:, 0:C, :]
            for d in range(max(lo, 1), hi):
                r = pltpu.make_async_remote_copy(
                    src_ref=ybig.at[:, d * C:(d + 1) * C, :],
                    dst_ref=ybuf.at[:, d],
                    send_sem=ysend.at[d], recv_sem=yrecv.at[d],
                    device_id=(lax.rem(my - d + N_DEV, N_DEV),),
                    device_id_type=MESH)
                r.start()
                yd.append(r)
        for r in yd:
            r.wait_recv()

        acc = None
        for e in range(E):
            yall = ybuf[e].reshape(R, D)
            part = lax.dot_general(
                selbig[e], yall, (((0,), (0,)), ((), ())),
                preferred_element_type=jnp.float32)
            acc = part if acc is None else acc + part
        out_ref[...] = acc

        for r in xd:
            r.wait_send()
        for r in yd:
            r.wait_send()

    a2d = assign.reshape(1, T)

    return pl.pallas_call(
        body,
        out_shape=jax.ShapeDtypeStruct((T, D), jnp.float32),
        in_specs=[pl.BlockSpec(memory_space=pltpu.VMEM)] * 4,
        out_specs=pl.BlockSpec(memory_space=pltpu.VMEM),
        scratch_shapes=[
            pltpu.VMEM((E, D, F), jnp.bfloat16),
            pltpu.VMEM((E, F, D), jnp.bfloat16),
            pltpu.VMEM((E, R, T), jnp.bfloat16),
            pltpu.VMEM((E, R, D), jnp.bfloat16),
            pltpu.VMEM((E, N_DEV, C, D), jnp.bfloat16),
            pltpu.VMEM((E, N_DEV, C, D), jnp.bfloat16),
            pltpu.VMEM((E, R, D), jnp.bfloat16),
            pltpu.SemaphoreType.DMA((N_DEV,)),
            pltpu.SemaphoreType.DMA((N_DEV,)),
            pltpu.SemaphoreType.DMA((N_DEV,)),
            pltpu.SemaphoreType.DMA((N_DEV,)),
        ],
        compiler_params=pltpu.CompilerParams(collective_id=0),
    )(x, a2d, W1, W2)


# device time: 43069 ns/iter; 1.0672x vs baseline; 1.0672x over previous
import jax
import jax.numpy as jnp
from jax import lax
from jax.experimental import pallas as pl
from jax.experimental.pallas import tpu as pltpu

N_DEV = 16
C = 32
MESH = pl.DeviceIdType.MESH


def kernel(x, assign, W1, W2):
    T, D = x.shape
    E, _, F = W1.shape
    R = N_DEV * C

    def body(x_ref, a_ref, w1b, w2b, out_ref,
             selbig, xcbig, xbuf, ybuf, ybig,
             xsend, xrecv, ysend, yrecv):
        my = lax.axis_index("i")

        barrier = pltpu.get_barrier_semaphore()
        for d in range(1, N_DEV):
            peer = lax.rem(my + d, N_DEV)
            pl.semaphore_signal(barrier, inc=1, device_id=(peer,),
                                device_id_type=MESH)
        pl.semaphore_wait(barrier, N_DEV - 1)

        a_row = a_ref[...]
        gids = lax.broadcasted_iota(jnp.int32, (2 * N_DEV, T), 0)
        onehot = (a_row == gids).astype(jnp.float32)
        ti = lax.broadcasted_iota(jnp.int32, (T, T), 0)
        tj = lax.broadcasted_iota(jnp.int32, (T, T), 1)
        ltri = (ti <= tj).astype(jnp.float32)
        cum = jnp.dot(onehot, ltri, preferred_element_type=jnp.float32)
        rank_row = jnp.sum(onehot * (cum - 1.0), axis=0, keepdims=True)

        rows_d = lax.broadcasted_iota(jnp.int32, (R, T), 0) // C
        rows_k = lax.broadcasted_iota(jnp.int32, (R, T), 0) % C
        for e in range(E):
            dst_expert = 2 * lax.rem(my + rows_d, N_DEV) + e
            sel = (a_row == dst_expert) & (rank_row == rows_k.astype(jnp.float32))
            selbig[e] = sel.astype(jnp.bfloat16)

        xb = x_ref[...].astype(jnp.bfloat16)
        for e in range(E):
            xcbig[e] = jnp.dot(selbig[e], xb,
                               preferred_element_type=jnp.float32
                               ).astype(jnp.bfloat16)

        xbuf[:, 0] = xcbig[:, 0:C, :]
        xd = []
        for d in range(1, N_DEV):
            r = pltpu.make_async_remote_copy(
                src_ref=xcbig.at[:, d * C:(d + 1) * C, :],
                dst_ref=xbuf.at[:, d],
                send_sem=xsend.at[d], recv_sem=xrecv.at[d],
                device_id=(lax.rem(my + d, N_DEV),), device_id_type=MESH)
            r.start()
            xd.append(r)
        for r in xd:
            r.wait_recv()

        for e in range(E):
            xall = xbuf[e].reshape(R, D)
            h1 = jnp.maximum(
                jnp.dot(xall, w1b[e], preferred_element_type=jnp.float32),
                0.0).astype(jnp.bfloat16)
            ybig[e] = jnp.dot(h1, w2b[e],
                              preferred_element_type=jnp.float32
                              ).astype(jnp.bfloat16)

        ybuf[:, 0] = ybig[:, 0:C, :]
        yd = []
        for d in range(1, N_DEV):
            r = pltpu.make_async_remote_copy(
                src_ref=ybig.at[:, d * C:(d + 1) * C, :],
                dst_ref=ybuf.at[:, d],
                send_sem=ysend.at[d], recv_sem=yrecv.at[d],
                device_id=(lax.rem(my - d + N_DEV, N_DEV),),
                device_id_type=MESH)
            r.start()
            yd.append(r)
        for r in yd:
            r.wait_recv()

        acc = None
        for e in range(E):
            yall = ybuf[e].reshape(R, D)
            part = lax.dot_general(
                selbig[e], yall, (((0,), (0,)), ((), ())),
                preferred_element_type=jnp.float32)
            acc = part if acc is None else acc + part
        out_ref[...] = acc

        for r in xd:
            r.wait_send()
        for r in yd:
            r.wait_send()

    a2d = assign.reshape(1, T)
    w1b = W1.astype(jnp.bfloat16)
    w2b = W2.astype(jnp.bfloat16)

    return pl.pallas_call(
        body,
        out_shape=jax.ShapeDtypeStruct((T, D), jnp.float32),
        in_specs=[pl.BlockSpec(memory_space=pltpu.VMEM)] * 4,
        out_specs=pl.BlockSpec(memory_space=pltpu.VMEM),
        scratch_shapes=[
            pltpu.VMEM((E, R, T), jnp.bfloat16),
            pltpu.VMEM((E, R, D), jnp.bfloat16),
            pltpu.VMEM((E, N_DEV, C, D), jnp.bfloat16),
            pltpu.VMEM((E, N_DEV, C, D), jnp.bfloat16),
            pltpu.VMEM((E, R, D), jnp.bfloat16),
            pltpu.SemaphoreType.DMA((N_DEV,)),
            pltpu.SemaphoreType.DMA((N_DEV,)),
            pltpu.SemaphoreType.DMA((N_DEV,)),
            pltpu.SemaphoreType.DMA((N_DEV,)),
        ],
        compiler_params=pltpu.CompilerParams(collective_id=0),
    )(x, a2d, w1b, w2b)
